# baseline (speedup 1.0000x reference)
import jax
import jax.numpy as jnp
from jax import lax
from jax.experimental import pallas as pl
from jax.experimental.pallas import tpu as pltpu
from jax.experimental.pallas import tpu_sc as plsc

B = 16384
D = 32
PK = 4
W = D * PK
NC = 2
NS = 16
NW = NC * NS
BPW = B // NW
L = 16
CH = 4
C = BPW // CH
EGC = C // L
BLK = 128

_IDX_NAMES = ("aUE", "aUC", "nAUE", "nAUC", "aBHE", "aBTE", "aBC",
              "nABHE", "nABTE", "nABC", "tUCC", "tUPC", "tBCC", "tBPC",
              "uniqE", "uniqUC", "uniqBC")
_SLOT = {n: i for i, n in enumerate(_IDX_NAMES)}
NIDX = len(_IDX_NAMES)

E_LINES = 1000000 // PK
C_LINES = 100000 // PK


def _relayout_body(et, uct, bcht, bctt, te, tuc, tbch, tbct,
                   eo, uco, bcho, bcto,
                   in_v, out_v, sem_in, sem_out):
    cid = lax.axis_index("c")
    sid = lax.axis_index("s")
    wid = sid * NC + cid

    lane = lax.iota(jnp.int32, L)
    qrow = []
    qcol = []
    for q in range(W // L):
        m = q * L + lane
        qrow.append(m % D)
        qcol.append(m // D)

    def do_table(src, dst, nlines):
        nfull = nlines // BLK
        tail = nlines - nfull * BLK
        trips = (nfull + NW - 1) // NW

        def blk_body(i, _):
            g = jnp.minimum(wid + i * NW, nfull - 1)
            g0 = g * BLK
            pltpu.async_copy(
                src.at[:, pl.ds(g0 * PK, BLK * PK)], in_v, sem_in).wait()

            def line(j, _):
                c4 = j * PK
                for q in range(W // L):
                    vals = plsc.load_gather(in_v, [qrow[q], qcol[q] + c4])
                    out_v[j, pl.ds(q * L, L)] = vals
                return 0
            lax.fori_loop(0, BLK, line, 0)
            pltpu.async_copy(
                out_v, dst.at[pl.ds(g0, BLK)], sem_out).wait()
            return 0
        lax.fori_loop(0, trips, blk_body, 0)

        if tail:
            @pl.when(wid == NW - 1)
            def _():
                pltpu.async_copy(
                    tails[id(dst)],
                    dst.at[pl.ds(nfull * BLK, tail)], sem_out).wait()

    tails = {id(eo): te, id(uco): tuc, id(bcho): tbch, id(bcto): tbct}
    do_table(et, eo, E_LINES)
    do_table(uct, uco, C_LINES)
    do_table(bcht, bcho, C_LINES)
    do_table(bctt, bcto, C_LINES)


E_TAIL = E_LINES % BLK
C_TAIL = C_LINES % BLK


def _relayout(entityT, ucT, bchT, bctT, te, tuc, tbch, tbct):
    mesh = plsc.VectorSubcoreMesh(core_axis_name="c", subcore_axis_name="s")
    f = pl.kernel(
        _relayout_body,
        out_type=(jax.ShapeDtypeStruct((E_LINES, W), jnp.float32),
                  jax.ShapeDtypeStruct((C_LINES, W), jnp.float32),
                  jax.ShapeDtypeStruct((C_LINES, W), jnp.float32),
                  jax.ShapeDtypeStruct((C_LINES, W), jnp.float32)),
        mesh=mesh,
        compiler_params=pltpu.CompilerParams(needs_layout_passes=False),
        scratch_types=[
            pltpu.VMEM((D, BLK * PK), jnp.float32),
            pltpu.VMEM((BLK, W), jnp.float32),
            pltpu.SemaphoreType.DMA,
            pltpu.SemaphoreType.DMA,
        ],
    )
    return f(entityT, ucT, bchT, bctT, te, tuc, tbch, tbct)


def _body(idx_hbm, marg_hbm, ent_hbm, uc_hbm, bch_hbm, bct_hbm,
          o0, o1, o2, o3, o4, o5, o6, o7, o8, o9, o10, o11, o12,
          idx_v, row_v, bufA0, bufA1, bufB0, bufB1,
          v0, v1, v2, v3, v4, v5, v6, v7, v8, v9, v10, v11, v12, marg_v,
          sem_idx, semA, semB, sem_out):
    out_v = [v0, v1, v2, v3, v4, v5, v6, v7, v8, v9, v10, v11, v12]
    cid = lax.axis_index("c")
    sid = lax.axis_index("s")
    wid = sid * NC + cid
    base = wid * BPW

    h0 = pltpu.async_copy(idx_hbm.at[wid], idx_v, sem_idx)
    h1 = pltpu.async_copy(marg_hbm, marg_v, sem_idx)
    h0.wait()
    h1.wait()

    lane = lax.iota(jnp.int32, L)

    def mkrow(g, _):
        r = g // EGC
        s = pl.ds((g % EGC) * L, L)
        row_v[r, s] = lax.shift_right_logical(idx_v[r, s], 2)
        return 0
    lax.fori_loop(0, NIDX * CH * EGC, mkrow, 0)

    tbl = {"E": ent_hbm, "UC": uc_hbm, "BCH": bch_hbm, "BCT": bct_hbm}

    passes = [
        ([("E", _SLOT["aUE"]), ("UC", _SLOT["aUC"])], "member", (0, "set")),
        ([("E", _SLOT["aBHE"]), ("BCH", _SLOT["aBC"])], "member", (1, "set")),
        ([("E", _SLOT["aBTE"]), ("BCT", _SLOT["aBC"])], "member", (1, "add")),
        ([("E", _SLOT["nAUE"]), ("UC", _SLOT["nAUC"])], "member", (2, "hinge")),
        ([("E", _SLOT["nABHE"]), ("BCH", _SLOT["nABC"])], "member", (3, "set")),
        ([("E", _SLOT["nABTE"]), ("BCT", _SLOT["nABC"])], "member", (3, "hinge_add")),
        ([("UC", _SLOT["tUCC"]), ("UC", _SLOT["tUPC"])], "hier", (4, 6, "u")),
        ([("BCH", _SLOT["tBCC"]), ("BCH", _SLOT["tBPC"])], "hier", (5, 7, "h")),
        ([("BCT", _SLOT["tBCC"]), ("BCT", _SLOT["tBPC"])], "hier", (5, 7, "t")),
        ([("E", _SLOT["uniqE"])], "norm", (8,)),
        ([("UC", _SLOT["uniqUC"])], "uniqc", (9, 11, "set")),
        ([("BCH", _SLOT["uniqBC"])], "uniqc", (10, 12, "set")),
        ([("BCT", _SLOT["uniqBC"])], "uniqc", (10, 12, "add")),
    ]

    stages = [(p, k) for p in range(len(passes)) for k in range(CH)]
    pairs = [(bufA0, bufA1), (bufB0, bufB1)]
    sems = [semA, semB]

    def issue(si):
        p, k = stages[si]
        gathers = passes[p][0]
        bufs = pairs[si % 2]
        sem = sems[si % 2]
        hs = []
        for (tk, j), buf in zip(gathers, bufs):
            hs.append(pltpu.async_copy(
                tbl[tk].at[row_v.at[j * CH + k]], buf, sem))
        return hs

    zero = jnp.zeros((L,), jnp.float32)

    def dim_loop(e_ref, c_ref, brow, epos, cpos, mode):
        if mode == "member":
            def db(d, acc):
                ge = plsc.load_gather(e_ref, [brow, epos + d])
                gc = plsc.load_gather(c_ref, [brow, cpos + d])
                t = (1.0 - gc) * ge
                return acc + t * t
            return lax.fori_loop(0, D, db, zero, unroll=4)
        if mode == "hier":
            def db(d, carry):
                a, dc, dp = carry
                gc = plsc.load_gather(e_ref, [brow, epos + d])
                gp = plsc.load_gather(c_ref, [brow, cpos + d])
                t = gc * (1.0 - gp)
                return (a + t * t, dc + jnp.abs(gc), dp + jnp.abs(gp))
            return lax.fori_loop(0, D, db, (zero, zero, zero), unroll=4)
        if mode == "norm":
            def db(d, acc):
                ge = plsc.load_gather(e_ref, [brow, epos + d])
                return acc + ge * ge
            return lax.fori_loop(0, D, db, zero, unroll=4)
        def db(d, carry):
            a, n = carry
            gc = plsc.load_gather(e_ref, [brow, epos + d])
            t = gc * (1.0 - gc)
            return (a + t * t, n + jnp.abs(gc))
        return lax.fori_loop(0, D, db, (zero, zero), unroll=4)

    def compute(si):
        p, k = stages[si]
        gathers, kind, args = passes[p]
        b0, b1 = pairs[si % 2]
        mvec = marg_v[...]
        js = [j for (_, j) in gathers]

        def outer(le, _):
            sl = pl.ds(k * C + le * L, L)
            brow = le * L + lane
            i0 = idx_v[js[0] * CH + k, pl.ds(le * L, L)]
            pos0 = (i0 & 3) * D
            if len(js) > 1:
                i1 = idx_v[js[1] * CH + k, pl.ds(le * L, L)]
                pos1 = (i1 & 3) * D
            else:
                pos1 = None
            if kind == "member":
                oi, op = args
                s = dim_loop(b0, b1, brow, pos0, pos1, "member")
                if op == "set":
                    out_v[oi][sl] = s
                elif op == "add":
                    out_v[oi][sl] = out_v[oi][sl] + s
                elif op == "hinge":
                    out_v[oi][sl] = jnp.maximum(mvec - s, 0.0)
                else:
                    out_v[oi][sl] = jnp.maximum(
                        mvec - (out_v[oi][sl] + s), 0.0)
            elif kind == "hier":
                ai, ci, part = args
                a, dc, dp = dim_loop(b0, b1, brow, pos0, pos1, "hier")
                if part == "u":
                    out_v[ai][sl] = a
                    out_v[ci][sl] = jnp.maximum(dc + 1.0 - dp, 0.0)
                elif part == "h":
                    out_v[ai][sl] = a
                    out_v[ci][sl] = dc - dp
                else:
                    out_v[ai][sl] = out_v[ai][sl] + a
                    out_v[ci][sl] = jnp.maximum(
                        out_v[ci][sl] + dc - dp + 1.0, 0.0)
            elif kind == "norm":
                (oi,) = args
                s = dim_loop(b0, None, brow, pos0, None, "norm")
                t = s - 1.0
                out_v[oi][sl] = t * t
            else:
                ai, ci, op = args
                a, n = dim_loop(b0, None, brow, pos0, None, "uniqc")
                h = jnp.maximum(1.0 - n, 0.0)
                if op == "set":
                    out_v[ai][sl] = a
                    out_v[ci][sl] = h
                else:
                    out_v[ai][sl] = out_v[ai][sl] + a
                    out_v[ci][sl] = out_v[ci][sl] + h
            return 0

        lax.fori_loop(0, EGC, outer, 0)

    outs = [o0, o1, o2, o3, o4, o5, o6, o7, o8, o9, o10, o11, o12]
    done_after = {0: 0, 1: 2, 2: 3, 3: 5, 4: 6, 6: 6, 5: 8, 7: 8,
                  8: 9, 9: 10, 11: 10, 10: 12, 12: 12}

    out_handles = []
    hs = issue(0)
    for si in range(len(stages)):
        nxt = issue(si + 1) if si + 1 < len(stages) else []
        for h in hs:
            h.wait()
        compute(si)
        hs = nxt
        p, k = stages[si]
        if k == CH - 1:
            for oi, after in done_after.items():
                if after == p:
                    out_handles.append(pltpu.async_copy(
                        out_v[oi], outs[oi].at[pl.ds(base, BPW)], sem_out))
    for h in out_handles:
        h.wait()


def kernel(aUE, aUC, nAUE, nAUC, aBHE, aBTE, aBC, nABHE, nABTE, nABC,
           tUCC, tUPC, tBCC, tBPC, uniqE, uniqUC, uniqBC,
           rdHUC, rdTUC, rdBC, nRdHUC, nRdTUC, lossMargin, device,
           entityEmbed, uConceptEmbed, bConceptHEmbed, bConceptTEmbed):
    idx_arrays = (aUE, aUC, nAUE, nAUC, aBHE, aBTE, aBC, nABHE, nABTE,
                  nABC, tUCC, tUPC, tBCC, tBPC, uniqE, uniqUC, uniqBC)
    idx_all = jnp.stack(
        [a.reshape(NW, CH, C) for a in idx_arrays],
        axis=1).reshape(NW, NIDX * CH, C)
    marg = jnp.broadcast_to(jnp.asarray(lossMargin, jnp.float32), (L,))

    ne = (E_LINES // BLK) * BLK * PK
    nc = (C_LINES // BLK) * BLK * PK
    ent_rm, uc_rm, bch_rm, bct_rm = _relayout(
        entityEmbed.T, uConceptEmbed.T, bConceptHEmbed.T, bConceptTEmbed.T,
        entityEmbed[ne:].reshape(-1, W),
        uConceptEmbed[nc:].reshape(-1, W),
        bConceptHEmbed[nc:].reshape(-1, W),
        bConceptTEmbed[nc:].reshape(-1, W))

    mesh = plsc.VectorSubcoreMesh(core_axis_name="c", subcore_axis_name="s")
    out_type = tuple(jax.ShapeDtypeStruct((B,), jnp.float32)
                     for _ in range(13))
    f = pl.kernel(
        _body,
        out_type=out_type,
        mesh=mesh,
        compiler_params=pltpu.CompilerParams(needs_layout_passes=False),
        scratch_types=(
            [pltpu.VMEM((NIDX * CH, C), jnp.int32),
             pltpu.VMEM((NIDX * CH, C), jnp.int32)]
            + [pltpu.VMEM((C, W), jnp.float32) for _ in range(4)]
            + [pltpu.VMEM((BPW,), jnp.float32) for _ in range(13)]
            + [pltpu.VMEM((L,), jnp.float32)]
            + [pltpu.SemaphoreType.DMA for _ in range(4)]
        ),
    )
    return f(idx_all, marg, ent_rm, uc_rm, bch_rm, bct_rm)


# speedup vs baseline: 1.3045x; 1.3045x over previous
import jax
import jax.numpy as jnp
from jax import lax
from jax.experimental import pallas as pl
from jax.experimental.pallas import tpu as pltpu
from jax.experimental.pallas import tpu_sc as plsc

B = 16384
D = 32
PK = 4
W = D * PK
NC = 2
NS = 16
NW = NC * NS
BPW = B // NW
L = 16
CH = 4
C = BPW // CH
EGC = C // L
BLK = 128

_IDX_NAMES = ("aUE", "aUC", "nAUE", "nAUC", "aBHE", "aBTE", "aBC",
              "nABHE", "nABTE", "nABC", "tUCC", "tUPC", "tBCC", "tBPC",
              "uniqE", "uniqUC", "uniqBC")
_SLOT = {n: i for i, n in enumerate(_IDX_NAMES)}
NIDX = len(_IDX_NAMES)

E_LINES = 1000000 // PK
C_LINES = 100000 // PK


def _relayout_body(et, uct, bcht, bctt, te, tuc, tbch, tbct,
                   eo, uco, bcho, bcto,
                   in0, in1, out0, out1,
                   semi0, semi1, semo0, semo1):
    cid = lax.axis_index("c")
    sid = lax.axis_index("s")
    wid = sid * NC + cid

    lane = lax.iota(jnp.int32, L)
    bases = []
    for c0 in range(0, BLK * PK, L):
        c = c0 + lane
        bases.append((c // PK) * W + (c % PK) * D)

    def comp(inb, outb):
        def dbody(d, _):
            for k in range(BLK * PK // L):
                v = inb[d, pl.ds(k * L, L)]
                plsc.store_scatter(outb, [bases[k] + d], v)
            return 0
        lax.fori_loop(0, D, dbody, 0)

    def do_table(src, dst1d, nlines, tail1d):
        nfull = nlines // BLK
        tail = nlines - nfull * BLK
        trips = (nfull + NW - 1) // NW
        trips2 = (trips + 1) // 2

        def cofs(i):
            g = jnp.minimum(wid + i * NW, nfull - 1)
            return g * (BLK * PK)

        def issue_in(i, buf, sem):
            pltpu.async_copy(
                src.at[:, pl.ds(cofs(i), BLK * PK)], buf, sem)

        def wait_in(buf, sem):
            pltpu.make_async_copy(
                src.at[:, pl.ds(0, BLK * PK)], buf, sem).wait()

        def issue_out(i, buf, sem):
            pltpu.async_copy(
                buf, dst1d.at[pl.ds(cofs(i) * (W // (BLK * PK) * BLK)
                                    if False else
                                    (jnp.minimum(wid + i * NW, nfull - 1)
                                     * (BLK * W)), BLK * W)], sem)

        def wait_out(buf, sem):
            pltpu.make_async_copy(
                buf, dst1d.at[pl.ds(0, BLK * W)], sem).wait()

        issue_in(0, in0, semi0)

        def body(t, _):
            i0 = 2 * t
            i1 = jnp.minimum(2 * t + 1, trips - 1)
            issue_in(i1, in1, semi1)
            wait_in(in0, semi0)

            @pl.when(t > 0)
            def _():
                wait_out(out0, semo0)
            comp(in0, out0)
            issue_out(i0, out0, semo0)

            issue_in(jnp.minimum(2 * t + 2, trips - 1), in0, semi0)
            wait_in(in1, semi1)

            @pl.when(t > 0)
            def _():
                wait_out(out1, semo1)
            comp(in1, out1)
            issue_out(i1, out1, semo1)
            return 0
        lax.fori_loop(0, trips2, body, 0)
        wait_in(in0, semi0)
        wait_out(out0, semo0)
        wait_out(out1, semo1)

        if tail:
            @pl.when(wid == NW - 1)
            def _():
                pltpu.async_copy(
                    tails[id(dst1d)],
                    dst1d.at[pl.ds(nfull * BLK * W, tail * W)], semo0).wait()

    tails = {id(eo): te, id(uco): tuc, id(bcho): tbch, id(bcto): tbct}
    do_table(et, eo, E_LINES, te)
    do_table(uct, uco, C_LINES, tuc)
    do_table(bcht, bcho, C_LINES, tbch)
    do_table(bctt, bcto, C_LINES, tbct)


E_TAIL = E_LINES % BLK
C_TAIL = C_LINES % BLK


def _relayout(entityT, ucT, bchT, bctT, te, tuc, tbch, tbct):
    mesh = plsc.VectorSubcoreMesh(core_axis_name="c", subcore_axis_name="s")
    f = pl.kernel(
        _relayout_body,
        out_type=(jax.ShapeDtypeStruct((E_LINES * W,), jnp.float32),
                  jax.ShapeDtypeStruct((C_LINES * W,), jnp.float32),
                  jax.ShapeDtypeStruct((C_LINES * W,), jnp.float32),
                  jax.ShapeDtypeStruct((C_LINES * W,), jnp.float32)),
        mesh=mesh,
        compiler_params=pltpu.CompilerParams(needs_layout_passes=False),
        scratch_types=[
            pltpu.VMEM((D, BLK * PK), jnp.float32),
            pltpu.VMEM((D, BLK * PK), jnp.float32),
            pltpu.VMEM((BLK * W,), jnp.float32),
            pltpu.VMEM((BLK * W,), jnp.float32),
            pltpu.SemaphoreType.DMA,
            pltpu.SemaphoreType.DMA,
            pltpu.SemaphoreType.DMA,
            pltpu.SemaphoreType.DMA,
        ],
    )
    return f(entityT, ucT, bchT, bctT, te, tuc, tbch, tbct)


def _body(idx_hbm, marg_hbm, ent_hbm, uc_hbm, bch_hbm, bct_hbm,
          o0, o1, o2, o3, o4, o5, o6, o7, o8, o9, o10, o11, o12,
          idx_v, row_v, bufA0, bufA1, bufB0, bufB1,
          v0, v1, v2, v3, v4, v5, v6, v7, v8, v9, v10, v11, v12, marg_v,
          sem_idx, semA, semB, sem_out):
    out_v = [v0, v1, v2, v3, v4, v5, v6, v7, v8, v9, v10, v11, v12]
    cid = lax.axis_index("c")
    sid = lax.axis_index("s")
    wid = sid * NC + cid
    base = wid * BPW

    h0 = pltpu.async_copy(idx_hbm.at[wid], idx_v, sem_idx)
    h1 = pltpu.async_copy(marg_hbm, marg_v, sem_idx)
    h0.wait()
    h1.wait()

    lane = lax.iota(jnp.int32, L)

    def mkrow(g, _):
        r = g // EGC
        s = pl.ds((g % EGC) * L, L)
        row_v[r, s] = lax.shift_right_logical(idx_v[r, s], 2)
        return 0
    lax.fori_loop(0, NIDX * CH * EGC, mkrow, 0)

    tbl = {"E": ent_hbm, "UC": uc_hbm, "BCH": bch_hbm, "BCT": bct_hbm}

    passes = [
        ([("E", _SLOT["aUE"]), ("UC", _SLOT["aUC"])], "member", (0, "set")),
        ([("E", _SLOT["aBHE"]), ("BCH", _SLOT["aBC"])], "member", (1, "set")),
        ([("E", _SLOT["aBTE"]), ("BCT", _SLOT["aBC"])], "member", (1, "add")),
        ([("E", _SLOT["nAUE"]), ("UC", _SLOT["nAUC"])], "member", (2, "hinge")),
        ([("E", _SLOT["nABHE"]), ("BCH", _SLOT["nABC"])], "member", (3, "set")),
        ([("E", _SLOT["nABTE"]), ("BCT", _SLOT["nABC"])], "member", (3, "hinge_add")),
        ([("UC", _SLOT["tUCC"]), ("UC", _SLOT["tUPC"])], "hier", (4, 6, "u")),
        ([("BCH", _SLOT["tBCC"]), ("BCH", _SLOT["tBPC"])], "hier", (5, 7, "h")),
        ([("BCT", _SLOT["tBCC"]), ("BCT", _SLOT["tBPC"])], "hier", (5, 7, "t")),
        ([("E", _SLOT["uniqE"])], "norm", (8,)),
        ([("UC", _SLOT["uniqUC"])], "uniqc", (9, 11, "set")),
        ([("BCH", _SLOT["uniqBC"])], "uniqc", (10, 12, "set")),
        ([("BCT", _SLOT["uniqBC"])], "uniqc", (10, 12, "add")),
    ]

    stages = [(p, k) for p in range(len(passes)) for k in range(CH)]
    pairs = [(bufA0, bufA1), (bufB0, bufB1)]
    sems = [semA, semB]

    def issue(si):
        p, k = stages[si]
        gathers = passes[p][0]
        bufs = pairs[si % 2]
        sem = sems[si % 2]
        hs = []
        for (tk, j), buf in zip(gathers, bufs):
            hs.append(pltpu.async_copy(
                tbl[tk].at[row_v.at[j * CH + k]], buf, sem))
        return hs

    zero = jnp.zeros((L,), jnp.float32)

    def dim_loop(e_ref, c_ref, brow, epos, cpos, mode):
        if mode == "member":
            def db(d, acc):
                ge = plsc.load_gather(e_ref, [brow, epos + d])
                gc = plsc.load_gather(c_ref, [brow, cpos + d])
                t = (1.0 - gc) * ge
                return acc + t * t
            return lax.fori_loop(0, D, db, zero, unroll=4)
        if mode == "hier":
            def db(d, carry):
                a, dc, dp = carry
                gc = plsc.load_gather(e_ref, [brow, epos + d])
                gp = plsc.load_gather(c_ref, [brow, cpos + d])
                t = gc * (1.0 - gp)
                return (a + t * t, dc + jnp.abs(gc), dp + jnp.abs(gp))
            return lax.fori_loop(0, D, db, (zero, zero, zero), unroll=4)
        if mode == "norm":
            def db(d, acc):
                ge = plsc.load_gather(e_ref, [brow, epos + d])
                return acc + ge * ge
            return lax.fori_loop(0, D, db, zero, unroll=4)
        def db(d, carry):
            a, n = carry
            gc = plsc.load_gather(e_ref, [brow, epos + d])
            t = gc * (1.0 - gc)
            return (a + t * t, n + jnp.abs(gc))
        return lax.fori_loop(0, D, db, (zero, zero), unroll=4)

    def compute(si):
        p, k = stages[si]
        gathers, kind, args = passes[p]
        b0, b1 = pairs[si % 2]
        mvec = marg_v[...]
        js = [j for (_, j) in gathers]

        def outer(le, _):
            sl = pl.ds(k * C + le * L, L)
            brow = le * L + lane
            i0 = idx_v[js[0] * CH + k, pl.ds(le * L, L)]
            pos0 = (i0 & 3) * D
            if len(js) > 1:
                i1 = idx_v[js[1] * CH + k, pl.ds(le * L, L)]
                pos1 = (i1 & 3) * D
            else:
                pos1 = None
            if kind == "member":
                oi, op = args
                s = dim_loop(b0, b1, brow, pos0, pos1, "member")
                if op == "set":
                    out_v[oi][sl] = s
                elif op == "add":
                    out_v[oi][sl] = out_v[oi][sl] + s
                elif op == "hinge":
                    out_v[oi][sl] = jnp.maximum(mvec - s, 0.0)
                else:
                    out_v[oi][sl] = jnp.maximum(
                        mvec - (out_v[oi][sl] + s), 0.0)
            elif kind == "hier":
                ai, ci, part = args
                a, dc, dp = dim_loop(b0, b1, brow, pos0, pos1, "hier")
                if part == "u":
                    out_v[ai][sl] = a
                    out_v[ci][sl] = jnp.maximum(dc + 1.0 - dp, 0.0)
                elif part == "h":
                    out_v[ai][sl] = a
                    out_v[ci][sl] = dc - dp
                else:
                    out_v[ai][sl] = out_v[ai][sl] + a
                    out_v[ci][sl] = jnp.maximum(
                        out_v[ci][sl] + dc - dp + 1.0, 0.0)
            elif kind == "norm":
                (oi,) = args
                s = dim_loop(b0, None, brow, pos0, None, "norm")
                t = s - 1.0
                out_v[oi][sl] = t * t
            else:
                ai, ci, op = args
                a, n = dim_loop(b0, None, brow, pos0, None, "uniqc")
                h = jnp.maximum(1.0 - n, 0.0)
                if op == "set":
                    out_v[ai][sl] = a
                    out_v[ci][sl] = h
                else:
                    out_v[ai][sl] = out_v[ai][sl] + a
                    out_v[ci][sl] = out_v[ci][sl] + h
            return 0

        lax.fori_loop(0, EGC, outer, 0)

    outs = [o0, o1, o2, o3, o4, o5, o6, o7, o8, o9, o10, o11, o12]
    done_after = {0: 0, 1: 2, 2: 3, 3: 5, 4: 6, 6: 6, 5: 8, 7: 8,
                  8: 9, 9: 10, 11: 10, 10: 12, 12: 12}

    out_handles = []
    hs = issue(0)
    for si in range(len(stages)):
        nxt = issue(si + 1) if si + 1 < len(stages) else []
        for h in hs:
            h.wait()
        compute(si)
        hs = nxt
        p, k = stages[si]
        if k == CH - 1:
            for oi, after in done_after.items():
                if after == p:
                    out_handles.append(pltpu.async_copy(
                        out_v[oi], outs[oi].at[pl.ds(base, BPW)], sem_out))
    for h in out_handles:
        h.wait()


def kernel(aUE, aUC, nAUE, nAUC, aBHE, aBTE, aBC, nABHE, nABTE, nABC,
           tUCC, tUPC, tBCC, tBPC, uniqE, uniqUC, uniqBC,
           rdHUC, rdTUC, rdBC, nRdHUC, nRdTUC, lossMargin, device,
           entityEmbed, uConceptEmbed, bConceptHEmbed, bConceptTEmbed):
    idx_arrays = (aUE, aUC, nAUE, nAUC, aBHE, aBTE, aBC, nABHE, nABTE,
                  nABC, tUCC, tUPC, tBCC, tBPC, uniqE, uniqUC, uniqBC)
    idx_all = jnp.stack(
        [a.reshape(NW, CH, C) for a in idx_arrays],
        axis=1).reshape(NW, NIDX * CH, C)
    marg = jnp.broadcast_to(jnp.asarray(lossMargin, jnp.float32), (L,))

    ne = (E_LINES // BLK) * BLK * PK
    nc = (C_LINES // BLK) * BLK * PK
    ent_1d, uc_1d, bch_1d, bct_1d = _relayout(
        entityEmbed.T, uConceptEmbed.T, bConceptHEmbed.T, bConceptTEmbed.T,
        entityEmbed[ne:].reshape(-1),
        uConceptEmbed[nc:].reshape(-1),
        bConceptHEmbed[nc:].reshape(-1),
        bConceptTEmbed[nc:].reshape(-1))
    ent_rm = ent_1d.reshape(E_LINES, W)
    uc_rm = uc_1d.reshape(C_LINES, W)
    bch_rm = bch_1d.reshape(C_LINES, W)
    bct_rm = bct_1d.reshape(C_LINES, W)

    mesh = plsc.VectorSubcoreMesh(core_axis_name="c", subcore_axis_name="s")
    out_type = tuple(jax.ShapeDtypeStruct((B,), jnp.float32)
                     for _ in range(13))
    f = pl.kernel(
        _body,
        out_type=out_type,
        mesh=mesh,
        compiler_params=pltpu.CompilerParams(needs_layout_passes=False),
        scratch_types=(
            [pltpu.VMEM((NIDX * CH, C), jnp.int32),
             pltpu.VMEM((NIDX * CH, C), jnp.int32)]
            + [pltpu.VMEM((C, W), jnp.float32) for _ in range(4)]
            + [pltpu.VMEM((BPW,), jnp.float32) for _ in range(13)]
            + [pltpu.VMEM((L,), jnp.float32)]
            + [pltpu.SemaphoreType.DMA for _ in range(4)]
        ),
    )
    return f(idx_all, marg, ent_rm, uc_rm, bch_rm, bct_rm)


# speedup vs baseline: 1.6919x; 1.2969x over previous
import jax
import jax.numpy as jnp
from jax import lax
from jax.experimental import pallas as pl
from jax.experimental.pallas import tpu as pltpu
from jax.experimental.pallas import tpu_sc as plsc

B = 16384
D = 32
PK = 4
W = D * PK
NC = 2
NS = 16
NW = NC * NS
BPW = B // NW
L = 16
CH = 4
C = BPW // CH
EGC = C // L
BLK = 128

_IDX_NAMES = ("aUE", "aUC", "nAUE", "nAUC", "aBHE", "aBTE", "aBC",
              "nABHE", "nABTE", "nABC", "tUCC", "tUPC", "tBCC", "tBPC",
              "uniqE", "uniqUC", "uniqBC")
_SLOT = {n: i for i, n in enumerate(_IDX_NAMES)}
NIDX = len(_IDX_NAMES)

E_LINES = 1000000 // PK
C_LINES = 100000 // PK


def _relayout_body(et, uct, bcht, bctt, te, tuc, tbch, tbct,
                   eo, uco, bcho, bcto,
                   in0, in1, out0, out1,
                   semi0, semi1, semo0, semo1):
    cid = lax.axis_index("c")
    sid = lax.axis_index("s")
    wid = sid * NC + cid

    lane = lax.iota(jnp.int32, L)
    bases = []
    for c0 in range(0, BLK * PK, L):
        c = c0 + lane
        bases.append((c // PK) * W + (c % PK) * D)

    def comp(inb, outb):
        @plsc.parallel_loop(0, D, unroll=2)
        def dbody(d):
            for k in range(BLK * PK // L):
                v = inb[d, pl.ds(k * L, L)]
                plsc.store_scatter(outb, [bases[k] + d], v)

    def do_table(src, dst1d, nlines, tail1d):
        nfull = nlines // BLK
        tail = nlines - nfull * BLK
        trips = (nfull + NW - 1) // NW
        trips2 = (trips + 1) // 2

        def cofs(i):
            g = jnp.minimum(wid + i * NW, nfull - 1)
            return g * (BLK * PK)

        def issue_in(i, buf, sem):
            pltpu.async_copy(
                src.at[:, pl.ds(cofs(i), BLK * PK)], buf, sem)

        def wait_in(buf, sem):
            pltpu.make_async_copy(
                src.at[:, pl.ds(0, BLK * PK)], buf, sem).wait()

        def issue_out(i, buf, sem):
            pltpu.async_copy(
                buf, dst1d.at[pl.ds(cofs(i) * (W // (BLK * PK) * BLK)
                                    if False else
                                    (jnp.minimum(wid + i * NW, nfull - 1)
                                     * (BLK * W)), BLK * W)], sem)

        def wait_out(buf, sem):
            pltpu.make_async_copy(
                buf, dst1d.at[pl.ds(0, BLK * W)], sem).wait()

        issue_in(0, in0, semi0)

        def body(t, _):
            i0 = 2 * t
            i1 = jnp.minimum(2 * t + 1, trips - 1)
            issue_in(i1, in1, semi1)
            wait_in(in0, semi0)

            @pl.when(t > 0)
            def _():
                wait_out(out0, semo0)
            comp(in0, out0)
            issue_out(i0, out0, semo0)

            issue_in(jnp.minimum(2 * t + 2, trips - 1), in0, semi0)
            wait_in(in1, semi1)

            @pl.when(t > 0)
            def _():
                wait_out(out1, semo1)
            comp(in1, out1)
            issue_out(i1, out1, semo1)
            return 0
        lax.fori_loop(0, trips2, body, 0)
        wait_in(in0, semi0)
        wait_out(out0, semo0)
        wait_out(out1, semo1)

        if tail:
            @pl.when(wid == NW - 1)
            def _():
                pltpu.async_copy(
                    tails[id(dst1d)],
                    dst1d.at[pl.ds(nfull * BLK * W, tail * W)], semo0).wait()

    tails = {id(eo): te, id(uco): tuc, id(bcho): tbch, id(bcto): tbct}
    do_table(et, eo, E_LINES, te)
    do_table(uct, uco, C_LINES, tuc)
    do_table(bcht, bcho, C_LINES, tbch)
    do_table(bctt, bcto, C_LINES, tbct)


E_TAIL = E_LINES % BLK
C_TAIL = C_LINES % BLK


def _relayout(entityT, ucT, bchT, bctT, te, tuc, tbch, tbct):
    mesh = plsc.VectorSubcoreMesh(core_axis_name="c", subcore_axis_name="s")
    f = pl.kernel(
        _relayout_body,
        out_type=(jax.ShapeDtypeStruct((E_LINES * W,), jnp.float32),
                  jax.ShapeDtypeStruct((C_LINES * W,), jnp.float32),
                  jax.ShapeDtypeStruct((C_LINES * W,), jnp.float32),
                  jax.ShapeDtypeStruct((C_LINES * W,), jnp.float32)),
        mesh=mesh,
        compiler_params=pltpu.CompilerParams(needs_layout_passes=False),
        scratch_types=[
            pltpu.VMEM((D, BLK * PK), jnp.float32),
            pltpu.VMEM((D, BLK * PK), jnp.float32),
            pltpu.VMEM((BLK * W,), jnp.float32),
            pltpu.VMEM((BLK * W,), jnp.float32),
            pltpu.SemaphoreType.DMA,
            pltpu.SemaphoreType.DMA,
            pltpu.SemaphoreType.DMA,
            pltpu.SemaphoreType.DMA,
        ],
    )
    return f(entityT, ucT, bchT, bctT, te, tuc, tbch, tbct)


def _body(idx_hbm, marg_hbm, ent_hbm, uc_hbm, bch_hbm, bct_hbm,
          o0, o1, o2, o3, o4, o5, o6, o7, o8, o9, o10, o11, o12,
          idx_v, row_v, bufA0, bufA1, bufB0, bufB1,
          v0, v1, v2, v3, v4, v5, v6, v7, v8, v9, v10, v11, v12, marg_v,
          sem_idx, semA, semB, sem_out):
    out_v = [v0, v1, v2, v3, v4, v5, v6, v7, v8, v9, v10, v11, v12]
    cid = lax.axis_index("c")
    sid = lax.axis_index("s")
    wid = sid * NC + cid
    base = wid * BPW

    h0 = pltpu.async_copy(idx_hbm.at[wid], idx_v, sem_idx)
    h1 = pltpu.async_copy(marg_hbm, marg_v, sem_idx)
    h0.wait()
    h1.wait()

    lane = lax.iota(jnp.int32, L)

    def mkrow(g, _):
        r = g // EGC
        s = pl.ds((g % EGC) * L, L)
        row_v[r, s] = lax.shift_right_logical(idx_v[r, s], 2)
        return 0
    lax.fori_loop(0, NIDX * CH * EGC, mkrow, 0)

    tbl = {"E": ent_hbm, "UC": uc_hbm, "BCH": bch_hbm, "BCT": bct_hbm}

    passes = [
        ([("E", _SLOT["aUE"]), ("UC", _SLOT["aUC"])], "member", (0, "set")),
        ([("E", _SLOT["aBHE"]), ("BCH", _SLOT["aBC"])], "member", (1, "set")),
        ([("E", _SLOT["aBTE"]), ("BCT", _SLOT["aBC"])], "member", (1, "add")),
        ([("E", _SLOT["nAUE"]), ("UC", _SLOT["nAUC"])], "member", (2, "hinge")),
        ([("E", _SLOT["nABHE"]), ("BCH", _SLOT["nABC"])], "member", (3, "set")),
        ([("E", _SLOT["nABTE"]), ("BCT", _SLOT["nABC"])], "member", (3, "hinge_add")),
        ([("UC", _SLOT["tUCC"]), ("UC", _SLOT["tUPC"])], "hier", (4, 6, "u")),
        ([("BCH", _SLOT["tBCC"]), ("BCH", _SLOT["tBPC"])], "hier", (5, 7, "h")),
        ([("BCT", _SLOT["tBCC"]), ("BCT", _SLOT["tBPC"])], "hier", (5, 7, "t")),
        ([("E", _SLOT["uniqE"])], "norm", (8,)),
        ([("UC", _SLOT["uniqUC"])], "uniqc", (9, 11, "set")),
        ([("BCH", _SLOT["uniqBC"])], "uniqc", (10, 12, "set")),
        ([("BCT", _SLOT["uniqBC"])], "uniqc", (10, 12, "add")),
    ]

    stages = [(p, k) for p in range(len(passes)) for k in range(CH)]
    pairs = [(bufA0, bufA1), (bufB0, bufB1)]
    sems = [semA, semB]

    def issue(si):
        p, k = stages[si]
        gathers = passes[p][0]
        bufs = pairs[si % 2]
        sem = sems[si % 2]
        hs = []
        for (tk, j), buf in zip(gathers, bufs):
            hs.append(pltpu.async_copy(
                tbl[tk].at[row_v.at[j * CH + k]], buf, sem))
        return hs

    zero = jnp.zeros((L,), jnp.float32)

    def dim_loop(e_ref, c_ref, brow, epos, cpos, mode):
        if mode == "member":
            def db(d, acc):
                ge = plsc.load_gather(e_ref, [brow, epos + d])
                gc = plsc.load_gather(c_ref, [brow, cpos + d])
                t = (1.0 - gc) * ge
                return acc + t * t
            return lax.fori_loop(0, D, db, zero, unroll=4)
        if mode == "hier":
            def db(d, carry):
                a, dc, dp = carry
                gc = plsc.load_gather(e_ref, [brow, epos + d])
                gp = plsc.load_gather(c_ref, [brow, cpos + d])
                t = gc * (1.0 - gp)
                return (a + t * t, dc + jnp.abs(gc), dp + jnp.abs(gp))
            return lax.fori_loop(0, D, db, (zero, zero, zero), unroll=4)
        if mode == "norm":
            def db(d, acc):
                ge = plsc.load_gather(e_ref, [brow, epos + d])
                return acc + ge * ge
            return lax.fori_loop(0, D, db, zero, unroll=4)
        def db(d, carry):
            a, n = carry
            gc = plsc.load_gather(e_ref, [brow, epos + d])
            t = gc * (1.0 - gc)
            return (a + t * t, n + jnp.abs(gc))
        return lax.fori_loop(0, D, db, (zero, zero), unroll=4)

    def compute(si):
        p, k = stages[si]
        gathers, kind, args = passes[p]
        b0, b1 = pairs[si % 2]
        mvec = marg_v[...]
        js = [j for (_, j) in gathers]

        def outer(le, _):
            sl = pl.ds(k * C + le * L, L)
            brow = le * L + lane
            i0 = idx_v[js[0] * CH + k, pl.ds(le * L, L)]
            pos0 = (i0 & 3) * D
            if len(js) > 1:
                i1 = idx_v[js[1] * CH + k, pl.ds(le * L, L)]
                pos1 = (i1 & 3) * D
            else:
                pos1 = None
            if kind == "member":
                oi, op = args
                s = dim_loop(b0, b1, brow, pos0, pos1, "member")
                if op == "set":
                    out_v[oi][sl] = s
                elif op == "add":
                    out_v[oi][sl] = out_v[oi][sl] + s
                elif op == "hinge":
                    out_v[oi][sl] = jnp.maximum(mvec - s, 0.0)
                else:
                    out_v[oi][sl] = jnp.maximum(
                        mvec - (out_v[oi][sl] + s), 0.0)
            elif kind == "hier":
                ai, ci, part = args
                a, dc, dp = dim_loop(b0, b1, brow, pos0, pos1, "hier")
                if part == "u":
                    out_v[ai][sl] = a
                    out_v[ci][sl] = jnp.maximum(dc + 1.0 - dp, 0.0)
                elif part == "h":
                    out_v[ai][sl] = a
                    out_v[ci][sl] = dc - dp
                else:
                    out_v[ai][sl] = out_v[ai][sl] + a
                    out_v[ci][sl] = jnp.maximum(
                        out_v[ci][sl] + dc - dp + 1.0, 0.0)
            elif kind == "norm":
                (oi,) = args
                s = dim_loop(b0, None, brow, pos0, None, "norm")
                t = s - 1.0
                out_v[oi][sl] = t * t
            else:
                ai, ci, op = args
                a, n = dim_loop(b0, None, brow, pos0, None, "uniqc")
                h = jnp.maximum(1.0 - n, 0.0)
                if op == "set":
                    out_v[ai][sl] = a
                    out_v[ci][sl] = h
                else:
                    out_v[ai][sl] = out_v[ai][sl] + a
                    out_v[ci][sl] = out_v[ci][sl] + h
            return 0

        lax.fori_loop(0, EGC, outer, 0)

    outs = [o0, o1, o2, o3, o4, o5, o6, o7, o8, o9, o10, o11, o12]
    done_after = {0: 0, 1: 2, 2: 3, 3: 5, 4: 6, 6: 6, 5: 8, 7: 8,
                  8: 9, 9: 10, 11: 10, 10: 12, 12: 12}

    out_handles = []
    hs = issue(0)
    for si in range(len(stages)):
        nxt = issue(si + 1) if si + 1 < len(stages) else []
        for h in hs:
            h.wait()
        compute(si)
        hs = nxt
        p, k = stages[si]
        if k == CH - 1:
            for oi, after in done_after.items():
                if after == p:
                    out_handles.append(pltpu.async_copy(
                        out_v[oi], outs[oi].at[pl.ds(base, BPW)], sem_out))
    for h in out_handles:
        h.wait()


def kernel(aUE, aUC, nAUE, nAUC, aBHE, aBTE, aBC, nABHE, nABTE, nABC,
           tUCC, tUPC, tBCC, tBPC, uniqE, uniqUC, uniqBC,
           rdHUC, rdTUC, rdBC, nRdHUC, nRdTUC, lossMargin, device,
           entityEmbed, uConceptEmbed, bConceptHEmbed, bConceptTEmbed):
    idx_arrays = (aUE, aUC, nAUE, nAUC, aBHE, aBTE, aBC, nABHE, nABTE,
                  nABC, tUCC, tUPC, tBCC, tBPC, uniqE, uniqUC, uniqBC)
    idx_all = jnp.stack(
        [a.reshape(NW, CH, C) for a in idx_arrays],
        axis=1).reshape(NW, NIDX * CH, C)
    marg = jnp.broadcast_to(jnp.asarray(lossMargin, jnp.float32), (L,))

    ne = (E_LINES // BLK) * BLK * PK
    nc = (C_LINES // BLK) * BLK * PK
    ent_1d, uc_1d, bch_1d, bct_1d = _relayout(
        entityEmbed.T, uConceptEmbed.T, bConceptHEmbed.T, bConceptTEmbed.T,
        entityEmbed[ne:].reshape(-1),
        uConceptEmbed[nc:].reshape(-1),
        bConceptHEmbed[nc:].reshape(-1),
        bConceptTEmbed[nc:].reshape(-1))
    ent_rm = ent_1d.reshape(E_LINES, W)
    uc_rm = uc_1d.reshape(C_LINES, W)
    bch_rm = bch_1d.reshape(C_LINES, W)
    bct_rm = bct_1d.reshape(C_LINES, W)

    mesh = plsc.VectorSubcoreMesh(core_axis_name="c", subcore_axis_name="s")
    out_type = tuple(jax.ShapeDtypeStruct((B,), jnp.float32)
                     for _ in range(13))
    f = pl.kernel(
        _body,
        out_type=out_type,
        mesh=mesh,
        compiler_params=pltpu.CompilerParams(needs_layout_passes=False),
        scratch_types=(
            [pltpu.VMEM((NIDX * CH, C), jnp.int32),
             pltpu.VMEM((NIDX * CH, C), jnp.int32)]
            + [pltpu.VMEM((C, W), jnp.float32) for _ in range(4)]
            + [pltpu.VMEM((BPW,), jnp.float32) for _ in range(13)]
            + [pltpu.VMEM((L,), jnp.float32)]
            + [pltpu.SemaphoreType.DMA for _ in range(4)]
        ),
    )
    return f(idx_all, marg, ent_rm, uc_rm, bch_rm, bct_rm)


# speedup vs baseline: 4.2477x; 2.5107x over previous
import jax
import jax.numpy as jnp
from jax import lax
from jax.experimental import pallas as pl
from jax.experimental.pallas import tpu as pltpu
from jax.experimental.pallas import tpu_sc as plsc

B = 16384
D = 32
PK = 4
W = D * PK
NC = 2
NS = 16
NW = NC * NS
BPW = B // NW
L = 16
CH = 4
C = BPW // CH
EGC = C // L
BLK = 128

_IDX_NAMES = ("aUE", "aUC", "nAUE", "nAUC", "aBHE", "aBTE", "aBC",
              "nABHE", "nABTE", "nABC", "tUCC", "tUPC", "tBCC", "tBPC",
              "uniqE", "uniqUC", "uniqBC")
_SLOT = {n: i for i, n in enumerate(_IDX_NAMES)}
NIDX = len(_IDX_NAMES)

E_LINES = 1000000 // PK
C_LINES = 100000 // PK


def _relayout_body(et, uct, bcht, bctt, te, tuc, tbch, tbct,
                   eo, uco, bcho, bcto,
                   in0, in1, out0, out1,
                   semi0, semi1, semo0, semo1):
    cid = lax.axis_index("c")
    sid = lax.axis_index("s")
    wid = sid * NC + cid

    lane = lax.iota(jnp.int32, L)
    bases = []
    for c0 in range(0, BLK * PK, L):
        c = c0 + lane
        bases.append((c // PK) * W + (c % PK))

    def comp(inb, outb):
        @plsc.parallel_loop(0, D, unroll=2)
        def dbody(d):
            d4 = d * PK
            for k in range(BLK * PK // L):
                v = inb[d, pl.ds(k * L, L)]
                plsc.store_scatter(outb, [bases[k] + d4], v)

    def do_table(src, dst1d, nlines, tail1d):
        nfull = nlines // BLK
        tail = nlines - nfull * BLK
        trips = (nfull + NW - 1) // NW
        trips2 = (trips + 1) // 2

        def cofs(i):
            g = jnp.minimum(wid + i * NW, nfull - 1)
            return g * (BLK * PK)

        def issue_in(i, buf, sem):
            pltpu.async_copy(
                src.at[:, pl.ds(cofs(i), BLK * PK)], buf, sem)

        def wait_in(buf, sem):
            pltpu.make_async_copy(
                src.at[:, pl.ds(0, BLK * PK)], buf, sem).wait()

        def issue_out(i, buf, sem):
            pltpu.async_copy(
                buf, dst1d.at[pl.ds(cofs(i) * (W // (BLK * PK) * BLK)
                                    if False else
                                    (jnp.minimum(wid + i * NW, nfull - 1)
                                     * (BLK * W)), BLK * W)], sem)

        def wait_out(buf, sem):
            pltpu.make_async_copy(
                buf, dst1d.at[pl.ds(0, BLK * W)], sem).wait()

        issue_in(0, in0, semi0)

        def body(t, _):
            i0 = 2 * t
            i1 = jnp.minimum(2 * t + 1, trips - 1)
            issue_in(i1, in1, semi1)
            wait_in(in0, semi0)

            @pl.when(t > 0)
            def _():
                wait_out(out0, semo0)
            comp(in0, out0)
            issue_out(i0, out0, semo0)

            issue_in(jnp.minimum(2 * t + 2, trips - 1), in0, semi0)
            wait_in(in1, semi1)

            @pl.when(t > 0)
            def _():
                wait_out(out1, semo1)
            comp(in1, out1)
            issue_out(i1, out1, semo1)
            return 0
        lax.fori_loop(0, trips2, body, 0)
        wait_in(in0, semi0)
        wait_out(out0, semo0)
        wait_out(out1, semo1)

        if tail:
            @pl.when(wid == NW - 1)
            def _():
                pltpu.async_copy(
                    tails[id(dst1d)],
                    dst1d.at[pl.ds(nfull * BLK * W, tail * W)], semo0).wait()

    tails = {id(eo): te, id(uco): tuc, id(bcho): tbch, id(bcto): tbct}
    do_table(et, eo, E_LINES, te)
    do_table(uct, uco, C_LINES, tuc)
    do_table(bcht, bcho, C_LINES, tbch)
    do_table(bctt, bcto, C_LINES, tbct)


E_TAIL = E_LINES % BLK
C_TAIL = C_LINES % BLK


def _relayout(entityT, ucT, bchT, bctT, te, tuc, tbch, tbct):
    mesh = plsc.VectorSubcoreMesh(core_axis_name="c", subcore_axis_name="s")
    f = pl.kernel(
        _relayout_body,
        out_type=(jax.ShapeDtypeStruct((E_LINES * W,), jnp.float32),
                  jax.ShapeDtypeStruct((C_LINES * W,), jnp.float32),
                  jax.ShapeDtypeStruct((C_LINES * W,), jnp.float32),
                  jax.ShapeDtypeStruct((C_LINES * W,), jnp.float32)),
        mesh=mesh,
        compiler_params=pltpu.CompilerParams(needs_layout_passes=False),
        scratch_types=[
            pltpu.VMEM((D, BLK * PK), jnp.float32),
            pltpu.VMEM((D, BLK * PK), jnp.float32),
            pltpu.VMEM((BLK * W,), jnp.float32),
            pltpu.VMEM((BLK * W,), jnp.float32),
            pltpu.SemaphoreType.DMA,
            pltpu.SemaphoreType.DMA,
            pltpu.SemaphoreType.DMA,
            pltpu.SemaphoreType.DMA,
        ],
    )
    return f(entityT, ucT, bchT, bctT, te, tuc, tbch, tbct)


def _body(idx_hbm, marg_hbm, ent_hbm, uc_hbm, bch_hbm, bct_hbm,
          o0, o1, o2, o3, o4, o5, o6, o7, o8, o9, o10, o11, o12,
          idx_v, row_v, bufA0, bufA1, bufB0, bufB1,
          v0, v1, v2, v3, v4, v5, v6, v7, v8, v9, v10, v11, v12, marg_v,
          sem_idx, semA, semB, sem_out):
    out_v = [v0, v1, v2, v3, v4, v5, v6, v7, v8, v9, v10, v11, v12]
    cid = lax.axis_index("c")
    sid = lax.axis_index("s")
    wid = sid * NC + cid
    base = wid * BPW

    h0 = pltpu.async_copy(idx_hbm.at[wid], idx_v, sem_idx)
    h1 = pltpu.async_copy(marg_hbm, marg_v, sem_idx)
    h0.wait()
    h1.wait()

    lane = lax.iota(jnp.int32, L)

    def mkrow(g, _):
        r = g // EGC
        s = pl.ds((g % EGC) * L, L)
        row_v[r, s] = lax.shift_right_logical(idx_v[r, s], 2)
        return 0
    lax.fori_loop(0, NIDX * CH * EGC, mkrow, 0)

    tbl = {"E": ent_hbm, "UC": uc_hbm, "BCH": bch_hbm, "BCT": bct_hbm}

    passes = [
        ([("E", _SLOT["aUE"]), ("UC", _SLOT["aUC"])], "member", (0, "set")),
        ([("E", _SLOT["aBHE"]), ("BCH", _SLOT["aBC"])], "member", (1, "set")),
        ([("E", _SLOT["aBTE"]), ("BCT", _SLOT["aBC"])], "member", (1, "add")),
        ([("E", _SLOT["nAUE"]), ("UC", _SLOT["nAUC"])], "member", (2, "hinge")),
        ([("E", _SLOT["nABHE"]), ("BCH", _SLOT["nABC"])], "member", (3, "set")),
        ([("E", _SLOT["nABTE"]), ("BCT", _SLOT["nABC"])], "member", (3, "hinge_add")),
        ([("UC", _SLOT["tUCC"]), ("UC", _SLOT["tUPC"])], "hier", (4, 6, "u")),
        ([("BCH", _SLOT["tBCC"]), ("BCH", _SLOT["tBPC"])], "hier", (5, 7, "h")),
        ([("BCT", _SLOT["tBCC"]), ("BCT", _SLOT["tBPC"])], "hier", (5, 7, "t")),
        ([("E", _SLOT["uniqE"])], "norm", (8,)),
        ([("UC", _SLOT["uniqUC"])], "uniqc", (9, 11, "set")),
        ([("BCH", _SLOT["uniqBC"])], "uniqc", (10, 12, "set")),
        ([("BCT", _SLOT["uniqBC"])], "uniqc", (10, 12, "add")),
    ]

    stages = [(p, k) for p in range(len(passes)) for k in range(CH)]
    pairs = [(bufA0, bufA1), (bufB0, bufB1)]
    sems = [semA, semB]

    def issue(si):
        p, k = stages[si]
        gathers = passes[p][0]
        bufs = pairs[si % 2]
        sem = sems[si % 2]
        hs = []
        for (tk, j), buf in zip(gathers, bufs):
            hs.append(pltpu.async_copy(
                tbl[tk].at[row_v.at[j * CH + k]], buf, sem))
        return hs

    zero = jnp.zeros((L,), jnp.float32)

    def dim_loop(e_ref, c_ref, brow, epos, cpos, mode):
        if mode == "member":
            def db(d, acc):
                ge = plsc.load_gather(e_ref, [brow, epos + d * PK])
                gc = plsc.load_gather(c_ref, [brow, cpos + d * PK])
                t = (1.0 - gc) * ge
                return acc + t * t
            return lax.fori_loop(0, D, db, zero, unroll=4)
        if mode == "hier":
            def db(d, carry):
                a, dc, dp = carry
                gc = plsc.load_gather(e_ref, [brow, epos + d * PK])
                gp = plsc.load_gather(c_ref, [brow, cpos + d * PK])
                t = gc * (1.0 - gp)
                return (a + t * t, dc + jnp.abs(gc), dp + jnp.abs(gp))
            return lax.fori_loop(0, D, db, (zero, zero, zero), unroll=4)
        if mode == "norm":
            def db(d, acc):
                ge = plsc.load_gather(e_ref, [brow, epos + d * PK])
                return acc + ge * ge
            return lax.fori_loop(0, D, db, zero, unroll=4)
        def db(d, carry):
            a, n = carry
            gc = plsc.load_gather(e_ref, [brow, epos + d * PK])
            t = gc * (1.0 - gc)
            return (a + t * t, n + jnp.abs(gc))
        return lax.fori_loop(0, D, db, (zero, zero), unroll=4)

    def compute(si):
        p, k = stages[si]
        gathers, kind, args = passes[p]
        b0, b1 = pairs[si % 2]
        mvec = marg_v[...]
        js = [j for (_, j) in gathers]

        def outer(le, _):
            sl = pl.ds(k * C + le * L, L)
            brow = le * L + lane
            i0 = idx_v[js[0] * CH + k, pl.ds(le * L, L)]
            pos0 = i0 & 3
            if len(js) > 1:
                i1 = idx_v[js[1] * CH + k, pl.ds(le * L, L)]
                pos1 = i1 & 3
            else:
                pos1 = None
            if kind == "member":
                oi, op = args
                s = dim_loop(b0, b1, brow, pos0, pos1, "member")
                if op == "set":
                    out_v[oi][sl] = s
                elif op == "add":
                    out_v[oi][sl] = out_v[oi][sl] + s
                elif op == "hinge":
                    out_v[oi][sl] = jnp.maximum(mvec - s, 0.0)
                else:
                    out_v[oi][sl] = jnp.maximum(
                        mvec - (out_v[oi][sl] + s), 0.0)
            elif kind == "hier":
                ai, ci, part = args
                a, dc, dp = dim_loop(b0, b1, brow, pos0, pos1, "hier")
                if part == "u":
                    out_v[ai][sl] = a
                    out_v[ci][sl] = jnp.maximum(dc + 1.0 - dp, 0.0)
                elif part == "h":
                    out_v[ai][sl] = a
                    out_v[ci][sl] = dc - dp
                else:
                    out_v[ai][sl] = out_v[ai][sl] + a
                    out_v[ci][sl] = jnp.maximum(
                        out_v[ci][sl] + dc - dp + 1.0, 0.0)
            elif kind == "norm":
                (oi,) = args
                s = dim_loop(b0, None, brow, pos0, None, "norm")
                t = s - 1.0
                out_v[oi][sl] = t * t
            else:
                ai, ci, op = args
                a, n = dim_loop(b0, None, brow, pos0, None, "uniqc")
                h = jnp.maximum(1.0 - n, 0.0)
                if op == "set":
                    out_v[ai][sl] = a
                    out_v[ci][sl] = h
                else:
                    out_v[ai][sl] = out_v[ai][sl] + a
                    out_v[ci][sl] = out_v[ci][sl] + h
            return 0

        lax.fori_loop(0, EGC, outer, 0)

    outs = [o0, o1, o2, o3, o4, o5, o6, o7, o8, o9, o10, o11, o12]
    done_after = {0: 0, 1: 2, 2: 3, 3: 5, 4: 6, 6: 6, 5: 8, 7: 8,
                  8: 9, 9: 10, 11: 10, 10: 12, 12: 12}

    out_handles = []
    hs = issue(0)
    for si in range(len(stages)):
        nxt = issue(si + 1) if si + 1 < len(stages) else []
        for h in hs:
            h.wait()
        compute(si)
        hs = nxt
        p, k = stages[si]
        if k == CH - 1:
            for oi, after in done_after.items():
                if after == p:
                    out_handles.append(pltpu.async_copy(
                        out_v[oi], outs[oi].at[pl.ds(base, BPW)], sem_out))
    for h in out_handles:
        h.wait()


def kernel(aUE, aUC, nAUE, nAUC, aBHE, aBTE, aBC, nABHE, nABTE, nABC,
           tUCC, tUPC, tBCC, tBPC, uniqE, uniqUC, uniqBC,
           rdHUC, rdTUC, rdBC, nRdHUC, nRdTUC, lossMargin, device,
           entityEmbed, uConceptEmbed, bConceptHEmbed, bConceptTEmbed):
    idx_arrays = (aUE, aUC, nAUE, nAUC, aBHE, aBTE, aBC, nABHE, nABTE,
                  nABC, tUCC, tUPC, tBCC, tBPC, uniqE, uniqUC, uniqBC)
    idx_all = jnp.stack(
        [a.reshape(NW, CH, C) for a in idx_arrays],
        axis=1).reshape(NW, NIDX * CH, C)
    marg = jnp.broadcast_to(jnp.asarray(lossMargin, jnp.float32), (L,))

    ne = (E_LINES // BLK) * BLK * PK
    nc = (C_LINES // BLK) * BLK * PK
    ent_1d, uc_1d, bch_1d, bct_1d = _relayout(
        entityEmbed.T, uConceptEmbed.T, bConceptHEmbed.T, bConceptTEmbed.T,
        entityEmbed[ne:].reshape(-1, PK, D).transpose(0, 2, 1).reshape(-1),
        uConceptEmbed[nc:].reshape(-1, PK, D).transpose(0, 2, 1).reshape(-1),
        bConceptHEmbed[nc:].reshape(-1, PK, D).transpose(0, 2, 1).reshape(-1),
        bConceptTEmbed[nc:].reshape(-1, PK, D).transpose(0, 2, 1).reshape(-1))
    ent_rm = ent_1d.reshape(E_LINES, W)
    uc_rm = uc_1d.reshape(C_LINES, W)
    bch_rm = bch_1d.reshape(C_LINES, W)
    bct_rm = bct_1d.reshape(C_LINES, W)

    mesh = plsc.VectorSubcoreMesh(core_axis_name="c", subcore_axis_name="s")
    out_type = tuple(jax.ShapeDtypeStruct((B,), jnp.float32)
                     for _ in range(13))
    f = pl.kernel(
        _body,
        out_type=out_type,
        mesh=mesh,
        compiler_params=pltpu.CompilerParams(needs_layout_passes=False),
        scratch_types=(
            [pltpu.VMEM((NIDX * CH, C), jnp.int32),
             pltpu.VMEM((NIDX * CH, C), jnp.int32)]
            + [pltpu.VMEM((C, W), jnp.float32) for _ in range(4)]
            + [pltpu.VMEM((BPW,), jnp.float32) for _ in range(13)]
            + [pltpu.VMEM((L,), jnp.float32)]
            + [pltpu.SemaphoreType.DMA for _ in range(4)]
        ),
    )
    return f(idx_all, marg, ent_rm, uc_rm, bch_rm, bct_rm)
